# Initial kernel scaffold; baseline (speedup 1.0000x reference)
import jax
import jax.numpy as jnp
from jax.experimental import pallas as pl


def kernel(x, emb, W0, a0, W_out, a_out, cls_a, attn_dense_W, attn_dense_b, frq_W, frq_b, cls_W, cls_b):
    raise NotImplementedError("write your pallas kernel here")


# speedup vs baseline: 1.2413x; 1.2413x over previous
import functools

import jax
import jax.numpy as jnp
from jax import lax
from jax.experimental import pallas as pl

_ALPHA = 0.2
_NEG = -1e30


def _lrelu(v):
    return jnp.where(v >= 0.0, v, _ALPHA * v)


def _elu(v):
    return jnp.where(v > 0.0, v, jnp.exp(jnp.minimum(v, 0.0)) - 1.0)


def _attend(active, e1, e2, Wh):
    e2m = jnp.max(jnp.where(active, e2, _NEG))
    M = _lrelu(e1 + e2m)
    E = _lrelu(e1[:, None] + e2[None, :])
    w = jnp.where(active[None, :], jnp.exp(jnp.minimum(E - M[:, None], 0.0)), 0.0)
    den = jnp.maximum(jnp.sum(w, axis=1), 1e-30)
    num = jnp.dot(w, Wh, preferred_element_type=jnp.float32)
    return _elu(num / den[:, None])


def _sample_body(H, D, P, x_ref, emb_ref, W0_ref, a0_ref, Wout_ref, aout_ref,
                 Wa_ref, ba_ref, lat_ref):
    xv = x_ref[0, 0, :]
    active = xv > 0.0
    feats = xv[:, None] * emb_ref[...]
    outs = []
    for h in range(H):
        Wh = jnp.dot(feats, W0_ref[h], preferred_element_type=jnp.float32)
        e1 = jnp.sum(Wh * a0_ref[h, :D][None, :], axis=1)
        e2 = jnp.sum(Wh * a0_ref[h, D:][None, :], axis=1)
        outs.append(_attend(active, e1, e2, Wh))
    feats2 = jnp.concatenate(outs, axis=1)
    Wh2 = jnp.dot(feats2, Wout_ref[...], preferred_element_type=jnp.float32)
    e1 = jnp.sum(Wh2 * aout_ref[:P][None, :], axis=1)
    e2 = jnp.sum(Wh2 * aout_ref[P:][None, :], axis=1)
    out2 = _attend(active, e1, e2, Wh2)
    g = xv[:, None] * out2
    T = lax.dot_general(g, Wa_ref[...], (((0,), (0,)), ((), ())),
                        preferred_element_type=jnp.float32)
    code = jnp.max(_elu(T + ba_ref[...][None, :]), axis=1)
    lat_ref[0, 0, :] = code


def _finale_body(K, B, H, D, P, x_ref, emb_ref, frqW_ref, frqb_ref, clsW_ref,
                 clsb_ref, clsa_ref, lat_ref, out_ref):
    for b in range(B):
        xc = jnp.clip(x_ref[b, 0, :] + x_ref[B + b, 0, :], 0.0, 1.0)
        embx = xc[:, None] * emb_ref[...]
        T = lax.dot_general(embx, frqW_ref[...], (((0,), (0,)), ((), ())),
                            preferred_element_type=jnp.float32)
        mod1 = jnp.max(_elu(T + frqb_ref[...][None, :]), axis=1)
        mod_cls = jnp.sum(mod1[:, None] * clsW_ref[...], axis=0) + clsb_ref[...]
        cls_code = _elu(mod_cls)
        lat_b = jnp.concatenate([lat_ref[b, :, :], lat_ref[B + b, :, :]], axis=0)
        acc = jnp.zeros((P,), jnp.float32)
        for h in range(H):
            e = _lrelu(jnp.sum(lat_b * clsa_ref[h, :P][None, :], axis=1)
                       + jnp.sum(cls_code * clsa_ref[h, P:]))
            m = jnp.max(e)
            wv = jnp.exp(e - m)
            attn = wv / jnp.sum(wv)
            acc = acc + jnp.sum(attn[:, None] * lat_b, axis=0)
        fused = acc / H
        out_ref[b, :] = _elu(fused + mod_cls)


def kernel(x, emb, W0, a0, W_out, a_out, cls_a, attn_dense_W, attn_dense_b,
           frq_W, frq_b, cls_W, cls_b):
    K, B, V = x.shape
    D = emb.shape[1]
    H = W0.shape[0]
    P = W_out.shape[1]
    Vp = ((V + 127) // 128) * 128

    pad = ((0, Vp - V), (0, 0))
    x_p = jnp.pad(x, ((0, 0), (0, 0), (0, Vp - V))).reshape(K * B, 1, Vp)
    emb_p = jnp.pad(emb, pad)
    Wa_p = jnp.pad(attn_dense_W, pad)
    frqW_p = jnp.pad(frq_W, pad)
    a0s = a0[..., 0]
    a_outs = a_out[:, 0]
    cls_as = cls_a[..., 0]

    latent = pl.pallas_call(
        functools.partial(_sample_body, H, D, P),
        grid=(K * B,),
        in_specs=[
            pl.BlockSpec((1, 1, Vp), lambda s: (s, 0, 0)),
            pl.BlockSpec((Vp, D), lambda s: (0, 0)),
            pl.BlockSpec((H, D, D), lambda s: (0, 0, 0)),
            pl.BlockSpec((H, 2 * D), lambda s: (0, 0)),
            pl.BlockSpec((D * H, P), lambda s: (0, 0)),
            pl.BlockSpec((2 * P,), lambda s: (0,)),
            pl.BlockSpec((Vp, D), lambda s: (0, 0)),
            pl.BlockSpec((D,), lambda s: (0,)),
        ],
        out_specs=pl.BlockSpec((1, 1, P), lambda s: (s, 0, 0)),
        out_shape=jax.ShapeDtypeStruct((K * B, 1, P), jnp.float32),
    )(x_p, emb_p, W0, a0s, W_out, a_outs, Wa_p, attn_dense_b)

    out = pl.pallas_call(
        functools.partial(_finale_body, K, B, H, D, P),
        out_shape=jax.ShapeDtypeStruct((B, P), jnp.float32),
    )(x_p, emb_p, frqW_p, frq_b, cls_W, cls_b, cls_as, latent)
    return out


# speedup vs baseline: 1.8971x; 1.5283x over previous
import functools

import jax
import jax.numpy as jnp
from jax import lax
from jax.experimental import pallas as pl

_ALPHA = 0.2
_NEG = -1e30


def _lrelu(v):
    return jnp.where(v >= 0.0, v, _ALPHA * v)


def _elu(v):
    return jnp.where(v > 0.0, v, jnp.exp(jnp.minimum(v, 0.0)) - 1.0)


def _attend(active, e1, e2, Whaug):
    e2m = jnp.max(jnp.where(active, e2, _NEG))
    t = e1 + e2m
    M = jnp.maximum(t, _ALPHA * t)
    u1 = jnp.exp(t - M)
    u2 = jnp.exp(_ALPHA * t - M)
    dv = e2 - e2m
    v1 = jnp.where(active, jnp.exp(dv), 0.0)
    v2 = jnp.where(active, jnp.exp(_ALPHA * dv), 0.0)
    w = jnp.maximum(u1[:, None] * v1[None, :], u2[:, None] * v2[None, :])
    nd = jnp.dot(w, Whaug, preferred_element_type=jnp.float32)
    den = jnp.maximum(nd[:, -1:], 1e-30)
    return _elu(nd[:, :-1] / den)


def _sample_body(H, D, P, x_ref, emb_ref, W0cat_ref, A12_ref, Wout_ref,
                 Aout_ref, Wa_ref, ba_ref, lat_ref):
    xv = x_ref[0, 0, :]
    active = xv > 0.0
    feats = xv[:, None] * emb_ref[...]
    ones = jnp.ones((feats.shape[0], 1), jnp.float32)
    WhAll = jnp.dot(feats, W0cat_ref[...], preferred_element_type=jnp.float32)
    E12 = jnp.dot(feats, A12_ref[...], preferred_element_type=jnp.float32)
    outs = []
    for h in range(H):
        Whaug = jnp.concatenate([WhAll[:, h * D:(h + 1) * D], ones], axis=1)
        outs.append(_attend(active, E12[:, h], E12[:, H + h], Whaug))
    feats2 = jnp.concatenate(outs, axis=1)
    Wh2aug = jnp.concatenate(
        [jnp.dot(feats2, Wout_ref[...], preferred_element_type=jnp.float32),
         ones], axis=1)
    E12_2 = jnp.dot(feats2, Aout_ref[...], preferred_element_type=jnp.float32)
    out2 = _attend(active, E12_2[:, 0], E12_2[:, 1], Wh2aug)
    g = xv[:, None] * out2
    T = lax.dot_general(g, Wa_ref[...], (((0,), (0,)), ((), ())),
                        preferred_element_type=jnp.float32)
    code = jnp.max(_elu(T + ba_ref[...][None, :]), axis=1)
    lat_ref[0, 0, :] = code


def _finale_body(K, B, H, D, P, x_ref, emb_ref, frqW_ref, frqb_ref, clsW_ref,
                 clsb_ref, clsa_ref, lat_ref, out_ref):
    for b in range(B):
        xc = jnp.clip(x_ref[b, 0, :] + x_ref[B + b, 0, :], 0.0, 1.0)
        embx = xc[:, None] * emb_ref[...]
        T = lax.dot_general(embx, frqW_ref[...], (((0,), (0,)), ((), ())),
                            preferred_element_type=jnp.float32)
        mod1 = jnp.max(_elu(T + frqb_ref[...][None, :]), axis=1)
        mod_cls = jnp.sum(mod1[:, None] * clsW_ref[...], axis=0) + clsb_ref[...]
        cls_code = _elu(mod_cls)
        lat_b = jnp.concatenate([lat_ref[b, :, :], lat_ref[B + b, :, :]], axis=0)
        acc = jnp.zeros((P,), jnp.float32)
        for h in range(H):
            e = _lrelu(jnp.sum(lat_b * clsa_ref[h, :P][None, :], axis=1)
                       + jnp.sum(cls_code * clsa_ref[h, P:]))
            m = jnp.max(e)
            wv = jnp.exp(e - m)
            attn = wv / jnp.sum(wv)
            acc = acc + jnp.sum(attn[:, None] * lat_b, axis=0)
        fused = acc / H
        out_ref[b, :] = _elu(fused + mod_cls)


def kernel(x, emb, W0, a0, W_out, a_out, cls_a, attn_dense_W, attn_dense_b,
           frq_W, frq_b, cls_W, cls_b):
    K, B, V = x.shape
    D = emb.shape[1]
    H = W0.shape[0]
    P = W_out.shape[1]
    Vp = ((V + 127) // 128) * 128

    pad = ((0, Vp - V), (0, 0))
    x_p = jnp.pad(x, ((0, 0), (0, 0), (0, Vp - V))).reshape(K * B, 1, Vp)
    emb_p = jnp.pad(emb, pad)
    Wa_p = jnp.pad(attn_dense_W, pad)
    frqW_p = jnp.pad(frq_W, pad)
    cls_as = cls_a[..., 0]
    W0cat = jnp.transpose(W0, (1, 0, 2)).reshape(D, H * D)
    A1 = jnp.stack([W0[h] @ a0[h, :D, 0] for h in range(H)], axis=1)
    A2 = jnp.stack([W0[h] @ a0[h, D:, 0] for h in range(H)], axis=1)
    A12 = jnp.concatenate([A1, A2], axis=1)
    Aout = jnp.stack([W_out @ a_out[:P, 0], W_out @ a_out[P:, 0]], axis=1)

    latent = pl.pallas_call(
        functools.partial(_sample_body, H, D, P),
        grid=(K * B,),
        in_specs=[
            pl.BlockSpec((1, 1, Vp), lambda s: (s, 0, 0)),
            pl.BlockSpec((Vp, D), lambda s: (0, 0)),
            pl.BlockSpec((D, H * D), lambda s: (0, 0)),
            pl.BlockSpec((D, 2 * H), lambda s: (0, 0)),
            pl.BlockSpec((D * H, P), lambda s: (0, 0)),
            pl.BlockSpec((D * H, 2), lambda s: (0, 0)),
            pl.BlockSpec((Vp, D), lambda s: (0, 0)),
            pl.BlockSpec((D,), lambda s: (0,)),
        ],
        out_specs=pl.BlockSpec((1, 1, P), lambda s: (s, 0, 0)),
        out_shape=jax.ShapeDtypeStruct((K * B, 1, P), jnp.float32),
    )(x_p, emb_p, W0cat, A12, W_out, Aout, Wa_p, attn_dense_b)

    out = pl.pallas_call(
        functools.partial(_finale_body, K, B, H, D, P),
        out_shape=jax.ShapeDtypeStruct((B, P), jnp.float32),
    )(x_p, emb_p, frqW_p, frq_b, cls_W, cls_b, cls_as, latent)
    return out
